# Initial kernel scaffold; baseline (speedup 1.0000x reference)
import jax
import jax.numpy as jnp
from jax.experimental import pallas as pl


def kernel(x, edge_index, edge_attr, eig, W_pre, b_pre, W_post, b_post):
    raise NotImplementedError("write your pallas kernel here")


# speedup vs baseline: 1.2244x; 1.2244x over previous
import functools

import jax
import jax.numpy as jnp
from jax.experimental import pallas as pl

N = 10000
E = 320000
D = 128
AVG_D = 3.5

EPI_BLK = 1000


def _epilogue_body(x_ref, v_ref, sg_ref, mx_ref, mn_ref, cnt_ref,
                   wx_ref, wid_ref, wamp_ref, watt_ref, bpre_ref, bpost_ref,
                   out_ref):
    c = cnt_ref[...]
    has = c > 0.0
    cc = jnp.maximum(c, 1.0)
    vb = v_ref[...] + bpre_ref[...]
    mean = jnp.where(has, sg_ref[...] / cc + vb, 0.0)
    mx = jnp.where(has, mx_ref[...] + vb, 0.0)
    mn = jnp.where(has, mn_ref[...] + vb, 0.0)
    logd = jnp.log(c + 1.0)
    s1 = logd / AVG_D
    s2 = AVG_D / jnp.where(logd > 0.0, logd, 1.0)
    aggs = jnp.concatenate([mean, mx, mn], axis=1)
    x = x_ref[...]
    acc = x + bpost_ref[...]
    acc += jnp.dot(x, wx_ref[...], preferred_element_type=jnp.float32)
    acc += jnp.dot(aggs, wid_ref[...], preferred_element_type=jnp.float32)
    acc += s1 * jnp.dot(aggs, wamp_ref[...], preferred_element_type=jnp.float32)
    acc += s2 * jnp.dot(aggs, watt_ref[...], preferred_element_type=jnp.float32)
    out_ref[...] = acc


def _epilogue(x, v, sg, mx, mn, cnt, W_post, b_pre, b_post):
    Wx = W_post[0:D]
    Wid = W_post[D:4 * D]
    Wamp = W_post[4 * D:7 * D]
    Watt = W_post[7 * D:10 * D]
    grid = (N // EPI_BLK,)
    row_spec = pl.BlockSpec((EPI_BLK, D), lambda i: (i, 0))
    cnt_spec = pl.BlockSpec((EPI_BLK, 1), lambda i: (i, 0))
    full = lambda shape: pl.BlockSpec(shape, lambda i: (0, 0))
    return pl.pallas_call(
        _epilogue_body,
        grid=grid,
        in_specs=[row_spec, row_spec, row_spec, row_spec, row_spec, cnt_spec,
                  full((D, D)), full((3 * D, D)), full((3 * D, D)),
                  full((3 * D, D)), full((1, D)), full((1, D))],
        out_specs=row_spec,
        out_shape=jax.ShapeDtypeStruct((N, D), jnp.float32),
    )(x, v, sg, mx, mn, cnt, Wx, Wid, Wamp, Watt,
      b_pre.reshape(1, D), b_post.reshape(1, D))


def kernel(x, edge_index, edge_attr, eig, W_pre, b_pre, W_post, b_post):
    src = edge_index[0]
    dst = edge_index[1]
    W1 = W_pre[0:D]
    W2 = W_pre[D:2 * D]
    W3 = W_pre[2 * D:]
    u = x @ W1
    v = x @ W2
    w = edge_attr @ W3
    g = jnp.take(u, src, axis=0) + w
    cnt = jax.ops.segment_sum(jnp.ones((E,), jnp.float32), dst, num_segments=N)
    sg = jax.ops.segment_sum(g, dst, num_segments=N)
    mx = jax.ops.segment_max(g, dst, num_segments=N)
    mn = -jax.ops.segment_max(-g, dst, num_segments=N)
    return _epilogue(x, v, sg, mx, mn, cnt.reshape(N, 1),
                     W_post, b_pre, b_post)
